# baseline (speedup 1.0000x reference)
import functools

import jax
import jax.numpy as jnp
from jax import lax
from jax.experimental import pallas as pl
from jax.experimental.pallas import tpu as pltpu
from jax.experimental.pallas import tpu_sc as plsc

N = 10000
D = 128
G = 64
E = 320000
NC = 2
NS = 16
CHUNK = 128
EPAD = 327680
ROWS = EPAD // CHUNK
RPT = ROWS // (NC * NS)
NROW = 10240
ZB = NROW // NS

_mesh = plsc.VectorSubcoreMesh(
    core_axis_name="c", subcore_axis_name="s", num_cores=NC, num_subcores=NS)


@functools.partial(
    pl.kernel,
    out_type=jax.ShapeDtypeStruct((NC, NROW), jnp.float32),
    mesh=_mesh,
    scratch_types=[
        pltpu.VMEM((RPT, CHUNK), jnp.int32),
        pltpu.VMEM((CHUNK,), jnp.float32),
        pltpu.VMEM((ZB,), jnp.float32),
        pltpu.VMEM_SHARED((NROW,), jnp.float32),
        pltpu.SemaphoreType.DMA,
    ],
)
def _deg_call(ei3_hbm, ones_hbm, zer1_hbm, out_hbm, dstbuf, ones_v, zer_v,
              acc, sem):
    c = lax.axis_index("c")
    s = lax.axis_index("s")
    pltpu.sync_copy(ones_hbm, ones_v)
    pltpu.sync_copy(zer1_hbm, zer_v)
    pltpu.sync_copy(zer_v, acc.at[pl.ds(s * ZB, ZB)])
    row0 = (c * NS + s) * RPT
    pltpu.sync_copy(ei3_hbm.at[1, pl.ds(row0, RPT)], dstbuf)
    plsc.subcore_barrier()

    def fire(j, carry):
        pltpu.async_copy(ones_v, acc.at[dstbuf.at[j]], sem, add=True)
        return carry

    def drain(j, carry):
        pltpu.make_async_copy(ones_v, acc.at[dstbuf.at[j]], sem).wait()
        return carry

    def stage(k, carry):
        lax.fori_loop(k * 16, (k + 1) * 16, fire, 0)
        lax.fori_loop(k * 16, (k + 1) * 16, drain, 0)
        return carry

    lax.fori_loop(0, RPT // 16, stage, 0)
    plsc.subcore_barrier()
    pltpu.sync_copy(acc.at[pl.ds(s * ZB, ZB)], out_hbm.at[c, pl.ds(s * ZB, ZB)])


@functools.partial(
    pl.kernel,
    out_type=jax.ShapeDtypeStruct((NC, NROW, D), jnp.float32),
    mesh=_mesh,
    scratch_types=[
        pltpu.VMEM((RPT // 2, CHUNK), jnp.int32),
        pltpu.VMEM((RPT // 2, CHUNK), jnp.int32),
        pltpu.VMEM((CHUNK, D), jnp.float32),
        pltpu.VMEM((CHUNK, D), jnp.float32),
        pltpu.VMEM_SHARED((NROW, D), jnp.float32),
        pltpu.SemaphoreType.DMA,
        pltpu.SemaphoreType.DMA,
        pltpu.SemaphoreType.DMA,
    ],
)
def _scatter_call(g_hbm, ei3_hbm, zrows_hbm, out_hbm,
                  srcbuf, dstbuf, rows_a, rows_b, acc, gsa, gsb, semz):
    c = lax.axis_index("c")
    s = lax.axis_index("s")
    row0 = (c * NS + s) * RPT

    for b in range(5):
        pltpu.async_copy(zrows_hbm, acc.at[pl.ds(s * ZB + b * 128, 128)], semz)
    pltpu.sync_copy(ei3_hbm.at[0, pl.ds(row0, RPT // 2)], srcbuf)
    pltpu.sync_copy(ei3_hbm.at[1, pl.ds(row0, RPT // 2)], dstbuf)
    pltpu.async_copy(g_hbm.at[srcbuf.at[0]], rows_a, gsa)
    for b in range(5):
        pltpu.make_async_copy(
            zrows_hbm, acc.at[pl.ds(s * ZB + b * 128, 128)], semz).wait()
    plsc.subcore_barrier()

    HH = RPT // 2

    def body(i, carry):
        j0 = 2 * i
        cpb = pltpu.async_copy(g_hbm.at[srcbuf.at[j0 + 1]], rows_b, gsb)
        pltpu.make_async_copy(g_hbm.at[srcbuf.at[j0]], rows_a, gsa).wait()
        pltpu.sync_copy(rows_a, acc.at[dstbuf.at[j0]], add=True)

        @pl.when(j0 + 2 < HH)
        def _():
            pltpu.async_copy(g_hbm.at[srcbuf.at[j0 + 2]], rows_a, gsa)

        cpb.wait()
        pltpu.sync_copy(rows_b, acc.at[dstbuf.at[j0 + 1]], add=True)
        return carry

    lax.fori_loop(0, HH // 2, body, 0)

    pltpu.sync_copy(ei3_hbm.at[0, pl.ds(row0 + HH, RPT // 2)], srcbuf)
    pltpu.sync_copy(ei3_hbm.at[1, pl.ds(row0 + HH, RPT // 2)], dstbuf)
    pltpu.async_copy(g_hbm.at[srcbuf.at[0]], rows_a, gsa)
    lax.fori_loop(0, HH // 2, body, 0)
    plsc.subcore_barrier()
    for b in range(5):
        off = s * ZB + b * 128
        pltpu.async_copy(acc.at[pl.ds(off, 128)], out_hbm.at[c, pl.ds(off, 128)],
                         semz)
    for b in range(5):
        off = s * ZB + b * 128
        pltpu.make_async_copy(
            acc.at[pl.ds(off, 128)], out_hbm.at[c, pl.ds(off, 128)], semz).wait()


BN = 2000


def _prep_body(degp_ref, x_ref, w1_ref, dinv_ref, g1_ref):
    deg = degp_ref[0] + degp_ref[1] + 1.0
    dinv = lax.rsqrt(jnp.maximum(deg, 1.0))
    t = jnp.dot(x_ref[...], w1_ref[...], preferred_element_type=jnp.float32)
    dinv_ref[...] = dinv
    g1_ref[...] = t * dinv


def _mid_body(sp_ref, g1_ref, dinv_ref, b1_ref, w2_ref, g2_ref):
    agg = sp_ref[0] + sp_ref[1] + g1_ref[...]
    h = jnp.maximum(dinv_ref[...] * agg + b1_ref[...], 0.0)
    g2_ref[...] = dinv_ref[...] * jnp.dot(
        h, w2_ref[...], preferred_element_type=jnp.float32)


def _pool_body(sp_ref, g2_ref, dinv_ref, b2_ref, batch_ref, out_ref,
               pooled, cnt):
    i = pl.program_id(0)
    agg = sp_ref[0] + sp_ref[1] + g2_ref[...]
    h = jnp.maximum(dinv_ref[...] * agg + b2_ref[...], 0.0)
    onehot = (batch_ref[...] == lax.broadcasted_iota(jnp.int32, (1, G), 1)
              ).astype(jnp.float32)
    p = lax.dot_general(onehot, h, (((0,), (0,)), ((), ())),
                        preferred_element_type=jnp.float32)
    q = lax.dot_general(onehot, jnp.ones_like(h), (((0,), (0,)), ((), ())),
                        preferred_element_type=jnp.float32)

    @pl.when(i == 0)
    def _():
        pooled[...] = jnp.zeros_like(pooled)
        cnt[...] = jnp.zeros_like(cnt)

    pooled[...] += p
    cnt[...] += q

    @pl.when(i == pl.num_programs(0) - 1)
    def _():
        out_ref[...] = pooled[...] / jnp.maximum(cnt[...], 1.0)


_prep = pl.pallas_call(
    _prep_body,
    grid=(N // BN,),
    in_specs=[
        pl.BlockSpec((NC, BN, 1), lambda i: (0, i, 0)),
        pl.BlockSpec((BN, D), lambda i: (i, 0)),
        pl.BlockSpec((D, D), lambda i: (0, 0)),
    ],
    out_specs=[
        pl.BlockSpec((BN, 1), lambda i: (i, 0)),
        pl.BlockSpec((BN, D), lambda i: (i, 0)),
    ],
    out_shape=[
        jax.ShapeDtypeStruct((N, 1), jnp.float32),
        jax.ShapeDtypeStruct((N, D), jnp.float32),
    ],
    compiler_params=pltpu.CompilerParams(dimension_semantics=("parallel",)),
)

_mid = pl.pallas_call(
    _mid_body,
    grid=(N // BN,),
    in_specs=[
        pl.BlockSpec((NC, BN, D), lambda i: (0, i, 0)),
        pl.BlockSpec((BN, D), lambda i: (i, 0)),
        pl.BlockSpec((BN, 1), lambda i: (i, 0)),
        pl.BlockSpec((1, D), lambda i: (0, 0)),
        pl.BlockSpec((D, D), lambda i: (0, 0)),
    ],
    out_specs=pl.BlockSpec((BN, D), lambda i: (i, 0)),
    out_shape=jax.ShapeDtypeStruct((N, D), jnp.float32),
    compiler_params=pltpu.CompilerParams(dimension_semantics=("parallel",)),
)

_pool = pl.pallas_call(
    _pool_body,
    grid=(N // BN,),
    in_specs=[
        pl.BlockSpec((NC, BN, D), lambda i: (0, i, 0)),
        pl.BlockSpec((BN, D), lambda i: (i, 0)),
        pl.BlockSpec((BN, 1), lambda i: (i, 0)),
        pl.BlockSpec((1, D), lambda i: (0, 0)),
        pl.BlockSpec((BN, 1), lambda i: (i, 0)),
    ],
    out_specs=pl.BlockSpec((G, D), lambda i: (0, 0)),
    out_shape=jax.ShapeDtypeStruct((G, D), jnp.float32),
    scratch_shapes=[
        pltpu.VMEM((G, D), jnp.float32),
        pltpu.VMEM((G, D), jnp.float32),
    ],
)


def kernel(x, edge_index, batch, W1, b1, W2, b2):
    pad = jnp.concatenate(
        [jnp.zeros((1, EPAD - E), jnp.int32),
         jnp.full((1, EPAD - E), N, jnp.int32)], axis=0)
    ei3 = jnp.concatenate([edge_index, pad], axis=1).reshape(2, ROWS, CHUNK)

    ones1 = jnp.ones((CHUNK,), jnp.float32)
    zer1 = jnp.zeros((ZB,), jnp.float32)
    zrows = jnp.zeros((128, D), jnp.float32)

    degp = _deg_call(ei3, ones1, zer1)
    degp3 = degp.reshape(NC, NROW, 1)
    dinv, g1 = _prep(degp3, x, W1)
    s1 = _scatter_call(g1, ei3, zrows)
    g2 = _mid(s1, g1, dinv, b1.reshape(1, D), W2)
    s2 = _scatter_call(g2, ei3, zrows)
    return _pool(s2, g2, dinv, b2.reshape(1, D), batch.reshape(N, 1))


# speedup vs baseline: 1.0041x; 1.0041x over previous
import functools

import jax
import jax.numpy as jnp
from jax import lax
from jax.experimental import pallas as pl
from jax.experimental.pallas import tpu as pltpu
from jax.experimental.pallas import tpu_sc as plsc

N = 10000
D = 128
G = 64
E = 320000
NC = 2
NS = 16
CHUNK = 128
EPAD = 327680
ROWS = EPAD // CHUNK
RPT = ROWS // (NC * NS)
NROW = 10240
ZB = NROW // NS

_mesh = plsc.VectorSubcoreMesh(
    core_axis_name="c", subcore_axis_name="s", num_cores=NC, num_subcores=NS)


@functools.partial(
    pl.kernel,
    out_type=jax.ShapeDtypeStruct((NC, NROW), jnp.float32),
    mesh=_mesh,
    scratch_types=[
        pltpu.VMEM((RPT, CHUNK), jnp.int32),
        pltpu.VMEM((CHUNK,), jnp.float32),
        pltpu.VMEM((ZB,), jnp.float32),
        pltpu.VMEM_SHARED((NROW,), jnp.float32),
        pltpu.SemaphoreType.DMA,
    ],
)
def _deg_call(ei3_hbm, ones_hbm, zer1_hbm, out_hbm, dstbuf, ones_v, zer_v,
              acc, sem):
    c = lax.axis_index("c")
    s = lax.axis_index("s")
    pltpu.sync_copy(ones_hbm, ones_v)
    pltpu.sync_copy(zer1_hbm, zer_v)
    pltpu.sync_copy(zer_v, acc.at[pl.ds(s * ZB, ZB)])
    row0 = (c * NS + s) * RPT
    pltpu.sync_copy(ei3_hbm.at[1, pl.ds(row0, RPT)], dstbuf)
    plsc.subcore_barrier()

    def fire(j, carry):
        pltpu.async_copy(ones_v, acc.at[dstbuf.at[j]], sem, add=True)
        return carry

    def drain(j, carry):
        pltpu.make_async_copy(ones_v, acc.at[dstbuf.at[j]], sem).wait()
        return carry

    def stage(k, carry):
        lax.fori_loop(k * 16, (k + 1) * 16, fire, 0)
        lax.fori_loop(k * 16, (k + 1) * 16, drain, 0)
        return carry

    lax.fori_loop(0, RPT // 16, stage, 0)
    plsc.subcore_barrier()
    pltpu.sync_copy(acc.at[pl.ds(s * ZB, ZB)], out_hbm.at[c, pl.ds(s * ZB, ZB)])


@functools.partial(
    pl.kernel,
    out_type=jax.ShapeDtypeStruct((NC, NROW, D), jnp.float32),
    mesh=_mesh,
    scratch_types=[
        pltpu.VMEM((RPT // 2, CHUNK), jnp.int32),
        pltpu.VMEM((RPT // 2, CHUNK), jnp.int32),
        pltpu.VMEM((CHUNK, D), jnp.float32),
        pltpu.VMEM((CHUNK, D), jnp.float32),
        pltpu.VMEM_SHARED((NROW, D), jnp.float32),
        pltpu.SemaphoreType.DMA,
        pltpu.SemaphoreType.DMA,
        pltpu.SemaphoreType.DMA,
    ],
)
def _scatter_call(g_hbm, ei3_hbm, zrows_hbm, out_hbm,
                  srcbuf, dstbuf, rows_a, rows_b, acc, gsa, gsb, semz):
    c = lax.axis_index("c")
    s = lax.axis_index("s")
    row0 = (c * NS + s) * RPT

    for b in range(5):
        pltpu.async_copy(zrows_hbm, acc.at[pl.ds(s * ZB + b * 128, 128)], semz)
    pltpu.sync_copy(ei3_hbm.at[0, pl.ds(row0, RPT // 2)], srcbuf)
    pltpu.sync_copy(ei3_hbm.at[1, pl.ds(row0, RPT // 2)], dstbuf)
    pltpu.async_copy(g_hbm.at[srcbuf.at[0]], rows_a, gsa)
    for b in range(5):
        pltpu.make_async_copy(
            zrows_hbm, acc.at[pl.ds(s * ZB + b * 128, 128)], semz).wait()
    plsc.subcore_barrier()

    HH = RPT // 2

    def body(i, carry):
        j0 = 2 * i
        cpb = pltpu.async_copy(g_hbm.at[srcbuf.at[j0 + 1]], rows_b, gsb)
        pltpu.make_async_copy(g_hbm.at[srcbuf.at[j0]], rows_a, gsa).wait()
        pltpu.sync_copy(rows_a, acc.at[dstbuf.at[j0]], add=True)

        @pl.when(j0 + 2 < HH)
        def _():
            pltpu.async_copy(g_hbm.at[srcbuf.at[j0 + 2]], rows_a, gsa)

        cpb.wait()
        pltpu.sync_copy(rows_b, acc.at[dstbuf.at[j0 + 1]], add=True)
        return carry

    lax.fori_loop(0, HH // 2, body, 0)

    pltpu.sync_copy(ei3_hbm.at[0, pl.ds(row0 + HH, RPT // 2)], srcbuf)
    pltpu.sync_copy(ei3_hbm.at[1, pl.ds(row0 + HH, RPT // 2)], dstbuf)
    pltpu.async_copy(g_hbm.at[srcbuf.at[0]], rows_a, gsa)
    lax.fori_loop(0, HH // 2, body, 0)
    plsc.subcore_barrier()
    for b in range(5):
        off = s * ZB + b * 128
        pltpu.async_copy(acc.at[pl.ds(off, 128)], out_hbm.at[c, pl.ds(off, 128)],
                         semz)
    for b in range(5):
        off = s * ZB + b * 128
        pltpu.make_async_copy(
            acc.at[pl.ds(off, 128)], out_hbm.at[c, pl.ds(off, 128)], semz).wait()


BN = 2000


def _prep_body(degp_ref, x_ref, w1_ref, dinv_ref, g1_ref):
    deg = degp_ref[0] + degp_ref[1] + 1.0
    dinv = lax.rsqrt(jnp.maximum(deg, 1.0))
    t = jnp.dot(x_ref[...], w1_ref[...], preferred_element_type=jnp.float32)
    dinv_ref[...] = dinv
    g1_ref[...] = t * dinv


def _mid_body(sp_ref, g1_ref, dinv_ref, b1_ref, w2_ref, g2_ref):
    agg = sp_ref[0] + sp_ref[1] + g1_ref[...]
    h = jnp.maximum(dinv_ref[...] * agg + b1_ref[...], 0.0)
    g2_ref[...] = dinv_ref[...] * jnp.dot(
        h, w2_ref[...], preferred_element_type=jnp.float32)


def _pool_body(sp_ref, g2_ref, dinv_ref, b2_ref, batch_ref, out_ref,
               pooled, cnt):
    i = pl.program_id(0)
    agg = sp_ref[0] + sp_ref[1] + g2_ref[...]
    h = jnp.maximum(dinv_ref[...] * agg + b2_ref[...], 0.0)
    onehot = (batch_ref[...] == lax.broadcasted_iota(jnp.int32, (1, G), 1)
              ).astype(jnp.float32)
    p = lax.dot_general(onehot, h, (((0,), (0,)), ((), ())),
                        preferred_element_type=jnp.float32)
    q = lax.dot_general(onehot, jnp.ones_like(h), (((0,), (0,)), ((), ())),
                        preferred_element_type=jnp.float32)

    @pl.when(i == 0)
    def _():
        pooled[...] = jnp.zeros_like(pooled)
        cnt[...] = jnp.zeros_like(cnt)

    pooled[...] += p
    cnt[...] += q

    @pl.when(i == pl.num_programs(0) - 1)
    def _():
        out_ref[...] = pooled[...] / jnp.maximum(cnt[...], 1.0)


_prep = pl.pallas_call(
    _prep_body,
    grid=(N // BN,),
    in_specs=[
        pl.BlockSpec((NC, BN, 1), lambda i: (0, i, 0)),
        pl.BlockSpec((BN, D), lambda i: (i, 0)),
        pl.BlockSpec((D, D), lambda i: (0, 0)),
    ],
    out_specs=[
        pl.BlockSpec((BN, 1), lambda i: (i, 0)),
        pl.BlockSpec((BN, D), lambda i: (i, 0)),
    ],
    out_shape=[
        jax.ShapeDtypeStruct((N, 1), jnp.float32),
        jax.ShapeDtypeStruct((N, D), jnp.float32),
    ],
    compiler_params=pltpu.CompilerParams(dimension_semantics=("parallel",)),
)

_mid = pl.pallas_call(
    _mid_body,
    grid=(N // BN,),
    in_specs=[
        pl.BlockSpec((NC, BN, D), lambda i: (0, i, 0)),
        pl.BlockSpec((BN, D), lambda i: (i, 0)),
        pl.BlockSpec((BN, 1), lambda i: (i, 0)),
        pl.BlockSpec((1, D), lambda i: (0, 0)),
        pl.BlockSpec((D, D), lambda i: (0, 0)),
    ],
    out_specs=pl.BlockSpec((BN, D), lambda i: (i, 0)),
    out_shape=jax.ShapeDtypeStruct((N, D), jnp.float32),
    compiler_params=pltpu.CompilerParams(dimension_semantics=("parallel",)),
)

_pool = pl.pallas_call(
    _pool_body,
    grid=(N // BN,),
    in_specs=[
        pl.BlockSpec((NC, BN, D), lambda i: (0, i, 0)),
        pl.BlockSpec((BN, D), lambda i: (i, 0)),
        pl.BlockSpec((BN, 1), lambda i: (i, 0)),
        pl.BlockSpec((1, D), lambda i: (0, 0)),
        pl.BlockSpec((BN, 1), lambda i: (i, 0)),
    ],
    out_specs=pl.BlockSpec((G, D), lambda i: (0, 0)),
    out_shape=jax.ShapeDtypeStruct((G, D), jnp.float32),
    scratch_shapes=[
        pltpu.VMEM((G, D), jnp.float32),
        pltpu.VMEM((G, D), jnp.float32),
    ],
)


def kernel(x, edge_index, batch, W1, b1, W2, b2):
    pad = jnp.concatenate(
        [jnp.zeros((1, EPAD - E), jnp.int32),
         (N + jnp.arange(EPAD - E, dtype=jnp.int32) % (NROW - N))[None, :]],
        axis=0)
    ei3 = jnp.concatenate([edge_index, pad], axis=1).reshape(2, ROWS, CHUNK)

    ones1 = jnp.ones((CHUNK,), jnp.float32)
    zer1 = jnp.zeros((ZB,), jnp.float32)
    zrows = jnp.zeros((128, D), jnp.float32)

    degp = _deg_call(ei3, ones1, zer1)
    degp3 = degp.reshape(NC, NROW, 1)
    dinv, g1 = _prep(degp3, x, W1)
    s1 = _scatter_call(g1, ei3, zrows)
    g2 = _mid(s1, g1, dinv, b1.reshape(1, D), W2)
    s2 = _scatter_call(g2, ei3, zrows)
    return _pool(s2, g2, dinv, b2.reshape(1, D), batch.reshape(N, 1))


# speedup vs baseline: 3.1859x; 3.1729x over previous
import functools

import jax
import jax.numpy as jnp
from jax import lax
from jax.experimental import pallas as pl
from jax.experimental.pallas import tpu as pltpu
from jax.experimental.pallas import tpu_sc as plsc

N = 10000
D = 128
G = 64
E = 320000
NC = 2
NS = 16
CHUNK = 128
EPAD = 327680
ROWS = EPAD // CHUNK
RPT = ROWS // (NC * NS)
NROW = 10240
ZB = NROW // NS

_mesh = plsc.VectorSubcoreMesh(
    core_axis_name="c", subcore_axis_name="s", num_cores=NC, num_subcores=NS)


@functools.partial(
    pl.kernel,
    out_type=jax.ShapeDtypeStruct((NC, NROW), jnp.float32),
    mesh=_mesh,
    scratch_types=[
        pltpu.VMEM((RPT, CHUNK), jnp.int32),
        pltpu.VMEM((CHUNK,), jnp.float32),
        pltpu.VMEM((ZB,), jnp.float32),
        pltpu.VMEM_SHARED((NROW,), jnp.float32),
        pltpu.SemaphoreType.DMA,
    ],
)
def _deg_call(ei3_hbm, ones_hbm, zer1_hbm, out_hbm, dstbuf, ones_v, zer_v,
              acc, sem):
    c = lax.axis_index("c")
    s = lax.axis_index("s")
    pltpu.sync_copy(ones_hbm, ones_v)
    pltpu.sync_copy(zer1_hbm, zer_v)
    pltpu.sync_copy(zer_v, acc.at[pl.ds(s * ZB, ZB)])
    row0 = (c * NS + s) * RPT
    pltpu.sync_copy(ei3_hbm.at[1, pl.ds(row0, RPT)], dstbuf)
    plsc.subcore_barrier()

    def fire(j, carry):
        pltpu.async_copy(ones_v, acc.at[dstbuf.at[j]], sem, add=True)
        return carry

    def drain(j, carry):
        pltpu.make_async_copy(ones_v, acc.at[dstbuf.at[j]], sem).wait()
        return carry

    def stage(k, carry):
        lax.fori_loop(k * 16, (k + 1) * 16, fire, 0)
        lax.fori_loop(k * 16, (k + 1) * 16, drain, 0)
        return carry

    lax.fori_loop(0, RPT // 16, stage, 0)
    plsc.subcore_barrier()
    pltpu.sync_copy(acc.at[pl.ds(s * ZB, ZB)], out_hbm.at[c, pl.ds(s * ZB, ZB)])


@functools.partial(
    pl.kernel,
    out_type=jax.ShapeDtypeStruct((NC, NROW, D), jnp.float32),
    mesh=_mesh,
    scratch_types=[
        pltpu.VMEM((RPT // 2, CHUNK), jnp.int32),
        pltpu.VMEM((RPT // 2, CHUNK), jnp.int32),
        pltpu.VMEM((CHUNK, D), jnp.float32),
        pltpu.VMEM((CHUNK, D), jnp.float32),
        pltpu.VMEM_SHARED((NROW, D), jnp.float32),
        pltpu.SemaphoreType.DMA,
        pltpu.SemaphoreType.DMA,
        pltpu.SemaphoreType.DMA,
    ],
)
def _scatter_call(g_hbm, ei3_hbm, zrows_hbm, out_hbm,
                  srcbuf, dstbuf, rows_a, rows_b, acc, gsa, gsb, semz):
    c = lax.axis_index("c")
    s = lax.axis_index("s")
    row0 = (c * NS + s) * RPT

    for b in range(5):
        pltpu.async_copy(zrows_hbm, acc.at[pl.ds(s * ZB + b * 128, 128)], semz)
    pltpu.sync_copy(ei3_hbm.at[0, pl.ds(row0, RPT // 2)], srcbuf)
    pltpu.sync_copy(ei3_hbm.at[1, pl.ds(row0, RPT // 2)], dstbuf)
    pltpu.async_copy(g_hbm.at[srcbuf.at[0]], rows_a, gsa)
    for b in range(5):
        pltpu.make_async_copy(
            zrows_hbm, acc.at[pl.ds(s * ZB + b * 128, 128)], semz).wait()
    plsc.subcore_barrier()

    HH = RPT // 2

    def body(i, carry):
        j0 = 2 * i
        cpb = pltpu.async_copy(g_hbm.at[srcbuf.at[j0 + 1]], rows_b, gsb)
        pltpu.make_async_copy(g_hbm.at[srcbuf.at[j0]], rows_a, gsa).wait()
        pltpu.sync_copy(rows_a, acc.at[dstbuf.at[j0]], add=True)

        @pl.when(j0 + 2 < HH)
        def _():
            pltpu.async_copy(g_hbm.at[srcbuf.at[j0 + 2]], rows_a, gsa)

        cpb.wait()
        pltpu.sync_copy(rows_b, acc.at[dstbuf.at[j0 + 1]], add=True)
        return carry

    lax.fori_loop(0, HH // 2, body, 0)

    pltpu.sync_copy(ei3_hbm.at[0, pl.ds(row0 + HH, RPT // 2)], srcbuf)
    pltpu.sync_copy(ei3_hbm.at[1, pl.ds(row0 + HH, RPT // 2)], dstbuf)
    pltpu.async_copy(g_hbm.at[srcbuf.at[0]], rows_a, gsa)
    lax.fori_loop(0, HH // 2, body, 0)
    plsc.subcore_barrier()
    for b in range(5):
        off = s * ZB + b * 128
        pltpu.async_copy(acc.at[pl.ds(off, 128)], out_hbm.at[c, pl.ds(off, 128)],
                         semz)
    for b in range(5):
        off = s * ZB + b * 128
        pltpu.make_async_copy(
            acc.at[pl.ds(off, 128)], out_hbm.at[c, pl.ds(off, 128)], semz).wait()


BN = 2000


def _prep_body(degp_ref, x_ref, w1_ref, dinv_ref, g1_ref):
    deg = degp_ref[0] + degp_ref[1] + 1.0
    dinv = lax.rsqrt(jnp.maximum(deg, 1.0))
    t = jnp.dot(x_ref[...], w1_ref[...], preferred_element_type=jnp.float32)
    dinv_ref[...] = dinv
    g1_ref[...] = t * dinv


def _mid_body(sp_ref, g1_ref, dinv_ref, b1_ref, w2_ref, g2_ref):
    agg = sp_ref[0] + sp_ref[1] + g1_ref[...]
    h = jnp.maximum(dinv_ref[...] * agg + b1_ref[...], 0.0)
    g2_ref[...] = dinv_ref[...] * jnp.dot(
        h, w2_ref[...], preferred_element_type=jnp.float32)


def _pool_body(sp_ref, g2_ref, dinv_ref, b2_ref, batch_ref, out_ref,
               pooled, cnt):
    i = pl.program_id(0)
    agg = sp_ref[0] + sp_ref[1] + g2_ref[...]
    h = jnp.maximum(dinv_ref[...] * agg + b2_ref[...], 0.0)
    onehot = (batch_ref[...] == lax.broadcasted_iota(jnp.int32, (1, G), 1)
              ).astype(jnp.float32)
    p = lax.dot_general(onehot, h, (((0,), (0,)), ((), ())),
                        preferred_element_type=jnp.float32)
    q = lax.dot_general(onehot, jnp.ones_like(h), (((0,), (0,)), ((), ())),
                        preferred_element_type=jnp.float32)

    @pl.when(i == 0)
    def _():
        pooled[...] = jnp.zeros_like(pooled)
        cnt[...] = jnp.zeros_like(cnt)

    pooled[...] += p
    cnt[...] += q

    @pl.when(i == pl.num_programs(0) - 1)
    def _():
        out_ref[...] = pooled[...] / jnp.maximum(cnt[...], 1.0)


_prep = pl.pallas_call(
    _prep_body,
    grid=(N // BN,),
    in_specs=[
        pl.BlockSpec((NC, BN, 1), lambda i: (0, i, 0)),
        pl.BlockSpec((BN, D), lambda i: (i, 0)),
        pl.BlockSpec((D, D), lambda i: (0, 0)),
    ],
    out_specs=[
        pl.BlockSpec((BN, 1), lambda i: (i, 0)),
        pl.BlockSpec((BN, D), lambda i: (i, 0)),
    ],
    out_shape=[
        jax.ShapeDtypeStruct((N, 1), jnp.float32),
        jax.ShapeDtypeStruct((N, D), jnp.float32),
    ],
    compiler_params=pltpu.CompilerParams(dimension_semantics=("parallel",)),
)

_mid = pl.pallas_call(
    _mid_body,
    grid=(N // BN,),
    in_specs=[
        pl.BlockSpec((NC, BN, D), lambda i: (0, i, 0)),
        pl.BlockSpec((BN, D), lambda i: (i, 0)),
        pl.BlockSpec((BN, 1), lambda i: (i, 0)),
        pl.BlockSpec((1, D), lambda i: (0, 0)),
        pl.BlockSpec((D, D), lambda i: (0, 0)),
    ],
    out_specs=pl.BlockSpec((BN, D), lambda i: (i, 0)),
    out_shape=jax.ShapeDtypeStruct((N, D), jnp.float32),
    compiler_params=pltpu.CompilerParams(dimension_semantics=("parallel",)),
)

_pool = pl.pallas_call(
    _pool_body,
    grid=(N // BN,),
    in_specs=[
        pl.BlockSpec((NC, BN, D), lambda i: (0, i, 0)),
        pl.BlockSpec((BN, D), lambda i: (i, 0)),
        pl.BlockSpec((BN, 1), lambda i: (i, 0)),
        pl.BlockSpec((1, D), lambda i: (0, 0)),
        pl.BlockSpec((BN, 1), lambda i: (i, 0)),
    ],
    out_specs=pl.BlockSpec((G, D), lambda i: (0, 0)),
    out_shape=jax.ShapeDtypeStruct((G, D), jnp.float32),
    scratch_shapes=[
        pltpu.VMEM((G, D), jnp.float32),
        pltpu.VMEM((G, D), jnp.float32),
    ],
)


def kernel(x, edge_index, batch, W1, b1, W2, b2):
    pidx = jnp.arange(EPAD - E, dtype=jnp.int32)
    pad = jnp.concatenate(
        [((pidx * 64) % N)[None, :],
         (N + pidx % (NROW - N))[None, :]], axis=0)
    ei3 = jnp.concatenate([edge_index, pad], axis=1).reshape(2, ROWS, CHUNK)

    ones1 = jnp.ones((CHUNK,), jnp.float32)
    zer1 = jnp.zeros((ZB,), jnp.float32)
    zrows = jnp.zeros((128, D), jnp.float32)

    degp = _deg_call(ei3, ones1, zer1)
    degp3 = degp.reshape(NC, NROW, 1)
    dinv, g1 = _prep(degp3, x, W1)
    s1 = _scatter_call(g1, ei3, zrows)
    g2 = _mid(s1, g1, dinv, b1.reshape(1, D), W2)
    s2 = _scatter_call(g2, ei3, zrows)
    return _pool(s2, g2, dinv, b2.reshape(1, D), batch.reshape(N, 1))
